# baseline (speedup 1.0000x reference)
import functools

import jax
import jax.numpy as jnp
from jax import lax
from jax.experimental import pallas as pl
from jax.experimental.pallas import tpu as pltpu
from jax.experimental.pallas import tpu_sc as plsc

N = 10000
K = 16
D = 128
H = 128

_NW = 32
_C = 120
_B = 2000

_SPLITS = (N,)


def _gather_rows(table, idx_grp, rpw, nch, clast):
    mesh = plsc.VectorSubcoreMesh(core_axis_name="c", subcore_axis_name="s")

    @functools.partial(
        pl.kernel,
        mesh=mesh,
        out_type=jax.ShapeDtypeStruct((_NW * rpw, D), jnp.float32),
        scratch_types=[
            pltpu.VMEM((nch, _C), jnp.int32),
            pltpu.VMEM((2, _C, D), jnp.float32),
            pltpu.SemaphoreType.DMA,
            pltpu.SemaphoreType.DMA,
        ],
    )
    def k(table_hbm, idx_hbm, out_hbm, idx_v, rows_v, sem0, sem1):
        sems = (sem0, sem1)
        wid = lax.axis_index("s") * 2 + lax.axis_index("c")
        base = wid * rpw
        pltpu.sync_copy(idx_hbm.at[wid], idx_v)

        def start(j, b):
            pltpu.make_async_copy(
                table_hbm.at[idx_v.at[j]], rows_v.at[b], sems[b]).start()

        def wait_g(j, b):
            pltpu.make_async_copy(
                table_hbm.at[idx_v.at[j]], rows_v.at[b], sems[b]).wait()

        def wb(j, b, n):
            pltpu.sync_copy(rows_v.at[b, pl.ds(0, n)],
                            out_hbm.at[pl.ds(base + j * _C, n)])

        start(0, 0)
        start(1, 1)

        def body(j, carry):
            for b in range(2):
                @pl.when(j % 2 == b)
                def _():
                    wait_g(j, b)
                    wb(j, b, _C)
                    start(j + 2, b)
            return carry

        lax.fori_loop(0, nch - 2, body, 0)
        wait_g(nch - 2, (nch - 2) % 2)
        wb(nch - 2, (nch - 2) % 2, _C)
        wait_g(nch - 1, (nch - 1) % 2)
        wb(nch - 1, (nch - 1) % 2, clast)

    return k(table, idx_grp)


def _sage_layer(m, xin, wihT, whhT, bvec, wselfT, wneighT, bout, act, nq):
    def body(m_ref, x_ref, wih_ref, whh_ref, b_ref, ws_ref, wn_ref, bo_ref,
             o_ref):
        wih = wih_ref[...]
        whh = whh_ref[...]
        bias = b_ref[...]
        h = jnp.zeros((_B, H), jnp.float32)
        c = jnp.zeros((_B, H), jnp.float32)
        for t in range(K):
            g = jnp.dot(m_ref[t], wih, preferred_element_type=jnp.float32)
            g = g + jnp.dot(h, whh, preferred_element_type=jnp.float32) + bias
            gi = 0.5 * jnp.tanh(g[:, 0 * H:1 * H]) + 0.5
            gf = 0.5 * jnp.tanh(g[:, 1 * H:2 * H]) + 0.5
            gg = jnp.tanh(g[:, 2 * H:3 * H])
            go = 0.5 * jnp.tanh(g[:, 3 * H:4 * H]) + 0.5
            c = gf * c + gi * gg
            h = go * jnp.tanh(c)
        out = (jnp.dot(x_ref[...], ws_ref[...],
                       preferred_element_type=jnp.float32)
               + jnp.dot(h, wn_ref[...], preferred_element_type=jnp.float32)
               + bo_ref[...])
        o_ref[...] = act(out)

    return pl.pallas_call(
        body,
        grid=(pl.cdiv(nq, _B),),
        in_specs=[
            pl.BlockSpec((K, _B, D), lambda i: (0, i, 0)),
            pl.BlockSpec((_B, D), lambda i: (i, 0)),
            pl.BlockSpec((D, 4 * H), lambda i: (0, 0)),
            pl.BlockSpec((H, 4 * H), lambda i: (0, 0)),
            pl.BlockSpec((1, 4 * H), lambda i: (0, 0)),
            pl.BlockSpec((D, H), lambda i: (0, 0)),
            pl.BlockSpec((H, H), lambda i: (0, 0)),
            pl.BlockSpec((1, H), lambda i: (0, 0)),
        ],
        out_specs=pl.BlockSpec((_B, H), lambda i: (i, 0)),
        out_shape=jax.ShapeDtypeStruct((nq, H), jnp.float32),
    )(m, xin, wihT, whhT, bvec, wselfT, wneighT, bout)


def _split_indices(src):
    out = []
    n0 = 0
    for nq in _SPLITS:
        rpw = nq * K // _NW
        nch = -(-rpw // _C)
        clast = rpw - (nch - 1) * _C
        seg = src.reshape(N, K)[n0:n0 + nq].T.reshape(_NW, rpw)
        pad = jnp.zeros((_NW, nch * _C - rpw), jnp.int32)
        idx = jnp.concatenate([seg, pad], axis=1).reshape(_NW, nch, _C)
        out.append((idx, rpw, nch, clast, n0, nq))
        n0 += nq
    return out


def kernel(x, edge_index, W_ih1, W_hh1, b_ih1, b_hh1, W_self1, W_neigh1,
           bias1, W_ih2, W_hh2, b_ih2, b_hh2, W_self2, W_neigh2, bias2):
    src = edge_index[0]
    groups = _split_indices(src)

    s = jnp.concatenate([
        jnp.full((H,), 0.5, jnp.float32),
        jnp.full((H,), 0.5, jnp.float32),
        jnp.ones((H,), jnp.float32),
        jnp.full((H,), 0.5, jnp.float32),
    ])

    def layer(table, xin, wih, whh, bih, bhh, wself, wneigh, bias, act):
        parts = []
        for idx, rpw, nch, clast, n0, nq in groups:
            m = _gather_rows(table, idx, rpw, nch, clast).reshape(K, nq, D)
            parts.append(_sage_layer(
                m, lax.dynamic_slice_in_dim(xin, n0, nq, 0),
                wih.T * s, whh.T * s, ((bih + bhh) * s).reshape(1, -1),
                wself.T, wneigh.T, bias.reshape(1, -1), act, nq))
        return jnp.concatenate(parts, axis=0)

    h1 = layer(x, x, W_ih1, W_hh1, b_ih1, b_hh1, W_self1, W_neigh1, bias1,
               jax.nn.relu)
    out = layer(h1, h1, W_ih2, W_hh2, b_ih2, b_hh2, W_self2, W_neigh2,
                bias2, jax.nn.sigmoid)
    return out


# speedup vs baseline: 1.0556x; 1.0556x over previous
import functools

import jax
import jax.numpy as jnp
from jax import lax
from jax.experimental import pallas as pl
from jax.experimental.pallas import tpu as pltpu
from jax.experimental.pallas import tpu_sc as plsc

N = 10000
K = 16
D = 128
H = 128

_NW = 32
_C = 120
_B = 1000

_SPLITS = (N,)


def _gather_rows(table, idx_grp, rpw, nch, clast):
    mesh = plsc.VectorSubcoreMesh(core_axis_name="c", subcore_axis_name="s")

    @functools.partial(
        pl.kernel,
        mesh=mesh,
        out_type=jax.ShapeDtypeStruct((_NW * rpw, D), jnp.float32),
        scratch_types=[
            pltpu.VMEM((nch, _C), jnp.int32),
            pltpu.VMEM((2, _C, D), jnp.float32),
            pltpu.SemaphoreType.DMA,
            pltpu.SemaphoreType.DMA,
        ],
    )
    def k(table_hbm, idx_hbm, out_hbm, idx_v, rows_v, sem0, sem1):
        sems = (sem0, sem1)
        wid = lax.axis_index("s") * 2 + lax.axis_index("c")
        base = wid * rpw
        pltpu.sync_copy(idx_hbm.at[wid], idx_v)

        def start(j, b):
            pltpu.make_async_copy(
                table_hbm.at[idx_v.at[j]], rows_v.at[b], sems[b]).start()

        def wait_g(j, b):
            pltpu.make_async_copy(
                table_hbm.at[idx_v.at[j]], rows_v.at[b], sems[b]).wait()

        def wb(j, b, n):
            pltpu.sync_copy(rows_v.at[b, pl.ds(0, n)],
                            out_hbm.at[pl.ds(base + j * _C, n)])

        start(0, 0)
        start(1, 1)

        def body(j, carry):
            for b in range(2):
                @pl.when(j % 2 == b)
                def _():
                    wait_g(j, b)
                    wb(j, b, _C)
                    start(j + 2, b)
            return carry

        lax.fori_loop(0, nch - 2, body, 0)
        wait_g(nch - 2, (nch - 2) % 2)
        wb(nch - 2, (nch - 2) % 2, _C)
        wait_g(nch - 1, (nch - 1) % 2)
        wb(nch - 1, (nch - 1) % 2, clast)

    return k(table, idx_grp)


def _sage_layer(m, xin, wihT, whhT, bvec, wselfT, wneighT, bout, act, nq):
    def body(m_ref, x_ref, wih_ref, whh_ref, b_ref, ws_ref, wn_ref, bo_ref,
             o_ref):
        wih = wih_ref[...]
        whh = whh_ref[...]
        bias = b_ref[...]
        h = jnp.zeros((_B, H), jnp.float32)
        c = jnp.zeros((_B, H), jnp.float32)
        for t in range(K):
            g = jnp.dot(m_ref[t], wih, preferred_element_type=jnp.float32)
            g = g + jnp.dot(h, whh, preferred_element_type=jnp.float32) + bias
            gi = 0.5 * jnp.tanh(g[:, 0 * H:1 * H]) + 0.5
            gf = 0.5 * jnp.tanh(g[:, 1 * H:2 * H]) + 0.5
            gg = jnp.tanh(g[:, 2 * H:3 * H])
            go = 0.5 * jnp.tanh(g[:, 3 * H:4 * H]) + 0.5
            c = gf * c + gi * gg
            h = go * jnp.tanh(c)
        out = (jnp.dot(x_ref[...], ws_ref[...],
                       preferred_element_type=jnp.float32)
               + jnp.dot(h, wn_ref[...], preferred_element_type=jnp.float32)
               + bo_ref[...])
        o_ref[...] = act(out)

    return pl.pallas_call(
        body,
        grid=(pl.cdiv(nq, _B),),
        in_specs=[
            pl.BlockSpec((K, _B, D), lambda i: (0, i, 0)),
            pl.BlockSpec((_B, D), lambda i: (i, 0)),
            pl.BlockSpec((D, 4 * H), lambda i: (0, 0)),
            pl.BlockSpec((H, 4 * H), lambda i: (0, 0)),
            pl.BlockSpec((1, 4 * H), lambda i: (0, 0)),
            pl.BlockSpec((D, H), lambda i: (0, 0)),
            pl.BlockSpec((H, H), lambda i: (0, 0)),
            pl.BlockSpec((1, H), lambda i: (0, 0)),
        ],
        out_specs=pl.BlockSpec((_B, H), lambda i: (i, 0)),
        out_shape=jax.ShapeDtypeStruct((nq, H), jnp.float32),
    )(m, xin, wihT, whhT, bvec, wselfT, wneighT, bout)


def _split_indices(src):
    out = []
    n0 = 0
    for nq in _SPLITS:
        rpw = nq * K // _NW
        nch = -(-rpw // _C)
        clast = rpw - (nch - 1) * _C
        seg = src.reshape(N, K)[n0:n0 + nq].T.reshape(_NW, rpw)
        pad = jnp.zeros((_NW, nch * _C - rpw), jnp.int32)
        idx = jnp.concatenate([seg, pad], axis=1).reshape(_NW, nch, _C)
        out.append((idx, rpw, nch, clast, n0, nq))
        n0 += nq
    return out


def kernel(x, edge_index, W_ih1, W_hh1, b_ih1, b_hh1, W_self1, W_neigh1,
           bias1, W_ih2, W_hh2, b_ih2, b_hh2, W_self2, W_neigh2, bias2):
    src = edge_index[0]
    groups = _split_indices(src)

    s = jnp.concatenate([
        jnp.full((H,), 0.5, jnp.float32),
        jnp.full((H,), 0.5, jnp.float32),
        jnp.ones((H,), jnp.float32),
        jnp.full((H,), 0.5, jnp.float32),
    ])

    def layer(table, xin, wih, whh, bih, bhh, wself, wneigh, bias, act):
        parts = []
        for idx, rpw, nch, clast, n0, nq in groups:
            m = _gather_rows(table, idx, rpw, nch, clast).reshape(K, nq, D)
            parts.append(_sage_layer(
                m, lax.dynamic_slice_in_dim(xin, n0, nq, 0),
                wih.T * s, whh.T * s, ((bih + bhh) * s).reshape(1, -1),
                wself.T, wneigh.T, bias.reshape(1, -1), act, nq))
        return jnp.concatenate(parts, axis=0)

    h1 = layer(x, x, W_ih1, W_hh1, b_ih1, b_hh1, W_self1, W_neigh1, bias1,
               jax.nn.relu)
    out = layer(h1, h1, W_ih2, W_hh2, b_ih2, b_hh2, W_self2, W_neigh2,
                bias2, jax.nn.sigmoid)
    return out


# speedup vs baseline: 1.0635x; 1.0075x over previous
import functools

import jax
import jax.numpy as jnp
from jax import lax
from jax.experimental import pallas as pl
from jax.experimental.pallas import tpu as pltpu
from jax.experimental.pallas import tpu_sc as plsc

N = 10000
K = 16
D = 128
H = 128

_NW = 32
_C = 120
_B = 1000

_SPLITS = (N,)


_NBUF = 6
_DEPTH = 3


def _gather_rows(table, idx_grp, rpw, nch, clast):
    assert nch % _NBUF == 0
    ngrp = nch // _NBUF
    mesh = plsc.VectorSubcoreMesh(core_axis_name="c", subcore_axis_name="s")

    @functools.partial(
        pl.kernel,
        mesh=mesh,
        out_type=jax.ShapeDtypeStruct((_NW * rpw, D), jnp.float32),
        scratch_types=[
            pltpu.VMEM((nch, _C), jnp.int32),
            pltpu.VMEM((_NBUF, _C, D), jnp.float32),
        ] + [pltpu.SemaphoreType.DMA] * (2 * _NBUF),
    )
    def k(table_hbm, idx_hbm, out_hbm, idx_v, rows_v, *sems):
        gs, ws = sems[:_NBUF], sems[_NBUF:]
        wid = lax.axis_index("s") * 2 + lax.axis_index("c")
        base = wid * rpw
        pltpu.sync_copy(idx_hbm.at[wid], idx_v)

        def gstart(j, b):
            pltpu.make_async_copy(
                table_hbm.at[idx_v.at[j]], rows_v.at[b], gs[b]).start()

        def gwait(j, b):
            pltpu.make_async_copy(
                table_hbm.at[idx_v.at[j]], rows_v.at[b], gs[b]).wait()

        def wdesc(j, b, n):
            return pltpu.make_async_copy(
                rows_v.at[b, pl.ds(0, n)],
                out_hbm.at[pl.ds(base + j * _C, n)], ws[b])

        for b in range(_DEPTH):
            gstart(b, b)

        def group(g, carry):
            for b in range(_NBUF):
                j = g * _NBUF + b
                gwait(j, b)
                if b == _NBUF - 1:
                    @pl.when(g < ngrp - 1)
                    def _():
                        wdesc(j, b, _C).start()

                    @pl.when(g == ngrp - 1)
                    def _():
                        wdesc(j, b, clast).start()
                else:
                    wdesc(j, b, _C).start()
                bn = (b + _DEPTH) % _NBUF
                jn = j + _DEPTH
                if b < _DEPTH:
                    @pl.when(g >= 1)
                    def _():
                        wdesc(0, bn, _C).wait()
                    gstart(jn, bn)
                else:
                    @pl.when(g < ngrp - 1)
                    def _():
                        wdesc(0, bn, _C).wait()
                        gstart(jn, bn)
            return carry

        lax.fori_loop(0, ngrp, group, 0)
        for b in range(_NBUF):
            wdesc(0, b, _C if b < _NBUF - 1 else clast).wait()

    return k(table, idx_grp)


def _sage_layer(m, xin, wihT, whhT, bvec, wselfT, wneighT, bout, act, nq):
    def body(m_ref, x_ref, wih_ref, whh_ref, b_ref, ws_ref, wn_ref, bo_ref,
             o_ref):
        wih = wih_ref[...]
        whh = whh_ref[...]
        bias = b_ref[...]
        h = jnp.zeros((_B, H), jnp.float32)
        c = jnp.zeros((_B, H), jnp.float32)
        for t in range(K):
            g = jnp.dot(m_ref[t], wih, preferred_element_type=jnp.float32)
            g = g + jnp.dot(h, whh, preferred_element_type=jnp.float32) + bias
            gi = 0.5 * jnp.tanh(g[:, 0 * H:1 * H]) + 0.5
            gf = 0.5 * jnp.tanh(g[:, 1 * H:2 * H]) + 0.5
            gg = jnp.tanh(g[:, 2 * H:3 * H])
            go = 0.5 * jnp.tanh(g[:, 3 * H:4 * H]) + 0.5
            c = gf * c + gi * gg
            h = go * jnp.tanh(c)
        out = (jnp.dot(x_ref[...], ws_ref[...],
                       preferred_element_type=jnp.float32)
               + jnp.dot(h, wn_ref[...], preferred_element_type=jnp.float32)
               + bo_ref[...])
        o_ref[...] = act(out)

    return pl.pallas_call(
        body,
        grid=(pl.cdiv(nq, _B),),
        in_specs=[
            pl.BlockSpec((K, _B, D), lambda i: (0, i, 0)),
            pl.BlockSpec((_B, D), lambda i: (i, 0)),
            pl.BlockSpec((D, 4 * H), lambda i: (0, 0)),
            pl.BlockSpec((H, 4 * H), lambda i: (0, 0)),
            pl.BlockSpec((1, 4 * H), lambda i: (0, 0)),
            pl.BlockSpec((D, H), lambda i: (0, 0)),
            pl.BlockSpec((H, H), lambda i: (0, 0)),
            pl.BlockSpec((1, H), lambda i: (0, 0)),
        ],
        out_specs=pl.BlockSpec((_B, H), lambda i: (i, 0)),
        out_shape=jax.ShapeDtypeStruct((nq, H), jnp.float32),
    )(m, xin, wihT, whhT, bvec, wselfT, wneighT, bout)


def _split_indices(src):
    out = []
    n0 = 0
    for nq in _SPLITS:
        rpw = nq * K // _NW
        nch = -(-rpw // _C)
        clast = rpw - (nch - 1) * _C
        seg = src.reshape(N, K)[n0:n0 + nq].T.reshape(_NW, rpw)
        pad = jnp.zeros((_NW, nch * _C - rpw), jnp.int32)
        idx = jnp.concatenate([seg, pad], axis=1).reshape(_NW, nch, _C)
        out.append((idx, rpw, nch, clast, n0, nq))
        n0 += nq
    return out


def kernel(x, edge_index, W_ih1, W_hh1, b_ih1, b_hh1, W_self1, W_neigh1,
           bias1, W_ih2, W_hh2, b_ih2, b_hh2, W_self2, W_neigh2, bias2):
    src = edge_index[0]
    groups = _split_indices(src)

    s = jnp.concatenate([
        jnp.full((H,), 0.5, jnp.float32),
        jnp.full((H,), 0.5, jnp.float32),
        jnp.ones((H,), jnp.float32),
        jnp.full((H,), 0.5, jnp.float32),
    ])

    def layer(table, xin, wih, whh, bih, bhh, wself, wneigh, bias, act):
        parts = []
        for idx, rpw, nch, clast, n0, nq in groups:
            m = _gather_rows(table, idx, rpw, nch, clast).reshape(K, nq, D)
            parts.append(_sage_layer(
                m, lax.dynamic_slice_in_dim(xin, n0, nq, 0),
                wih.T * s, whh.T * s, ((bih + bhh) * s).reshape(1, -1),
                wself.T, wneigh.T, bias.reshape(1, -1), act, nq))
        return jnp.concatenate(parts, axis=0)

    h1 = layer(x, x, W_ih1, W_hh1, b_ih1, b_hh1, W_self1, W_neigh1, bias1,
               jax.nn.relu)
    out = layer(h1, h1, W_ih2, W_hh2, b_ih2, b_hh2, W_self2, W_neigh2,
                bias2, jax.nn.sigmoid)
    return out
